# baseline (speedup 1.0000x reference)
import functools

import jax
import jax.numpy as jnp
from jax.experimental import pallas as pl
from jax.experimental.pallas import tpu as pltpu
from jax.experimental.pallas import tpu_sc as plsc

N = 5000
NUM_CLASSES = 20
NP = 5120
L = 16
NTILES = 32
NCHUNKS = NP // L
NROWS = 40
THRESH = 0.5
BLK = 1000


def _cls_kernel(s_ref, cls_ref, ssel_ref, key_ref):
    s = s_ref[...]
    m = jnp.max(s, axis=1, keepdims=True)
    lane = jax.lax.broadcasted_iota(jnp.int32, s.shape, 1)
    idx = jnp.min(jnp.where(s == m, lane, NUM_CLASSES), axis=1, keepdims=True)
    cls_ref[...] = idx
    ssel_ref[...] = m
    key_ref[...] = idx.astype(jnp.float32) * 2.0 - m


def _sc_nms(bh, ordh, sth, cnth, out_ref,
            bfv, ordv, stv, cntv,
            lx1, ly1, lx2, ly2, lar, suppv, lidx2, kval2):
    wid = jax.lax.axis_index("s") * 2 + jax.lax.axis_index("c")
    iota = jax.lax.iota(jnp.int32, L)
    dump = N + wid

    def fill_body(k, _):
        r = k // 8
        c = (k % 8) * L
        lidx2[r, pl.ds(c, L)] = jnp.full((L,), dump, jnp.int32)
        return 0

    jax.lax.fori_loop(0, NCHUNKS, fill_body, 0)

    pltpu.sync_copy(bh, bfv)
    pltpu.sync_copy(ordh, ordv)
    pltpu.sync_copy(sth, stv)
    pltpu.sync_copy(cnth, cntv)

    def sload(ref, i):
        v = plsc.load_gather(ref, [jnp.full((L,), i, jnp.int32)])
        return v[0]

    start = sload(stv, wid)
    n = sload(cntv, wid)
    nch = (n + L - 1) // L

    def gather_body(k, _):
        p16 = jnp.full((L,), start + k * L, jnp.int32) + iota
        idx16 = plsc.load_gather(ordv, [p16])
        lane_pos = jnp.full((L,), k * L, jnp.int32) + iota
        r = k // 8
        c = (k % 8) * L
        lidx2[r, pl.ds(c, L)] = jnp.where(lane_pos < n, idx16, dump)
        i4 = idx16 * 4
        a = plsc.load_gather(bfv, [i4])
        b = plsc.load_gather(bfv, [i4 + 1])
        cc = plsc.load_gather(bfv, [i4 + 2])
        d = plsc.load_gather(bfv, [i4 + 3])
        lx1[pl.ds(k * L, L)] = a
        ly1[pl.ds(k * L, L)] = b
        lx2[pl.ds(k * L, L)] = cc
        ly2[pl.ds(k * L, L)] = d
        lar[pl.ds(k * L, L)] = (cc - a) * (d - b)
        suppv[pl.ds(k * L, L)] = jnp.zeros((L,), jnp.float32)
        return 0

    jax.lax.fori_loop(0, nch, gather_body, 0)

    def outer(i, _):
        ii = jnp.full((L,), i, jnp.int32)
        si = plsc.load_gather(suppv, [ii])
        act = si[0] == 0.0

        @pl.when(act)
        def _sweep():
            x1i = plsc.load_gather(lx1, [ii])
            y1i = plsc.load_gather(ly1, [ii])
            x2i = plsc.load_gather(lx2, [ii])
            y2i = plsc.load_gather(ly2, [ii])
            ai = plsc.load_gather(lar, [ii])

            def overlap(b0):
                xx1 = jnp.maximum(lx1[pl.ds(b0, L)], x1i)
                yy1 = jnp.maximum(ly1[pl.ds(b0, L)], y1i)
                xx2 = jnp.minimum(lx2[pl.ds(b0, L)], x2i)
                yy2 = jnp.minimum(ly2[pl.ds(b0, L)], y2i)
                w = jnp.maximum(1e-10, xx2 - xx1)
                h = jnp.maximum(1e-10, yy2 - yy1)
                inter = w * h
                denom = (ai + lar[pl.ds(b0, L)]) - inter
                return (inter > THRESH * denom) & (denom >= 0.0)

            k0 = i // L
            b0 = k0 * L
            pos = jnp.full((L,), b0, jnp.int32) + iota
            ns0 = overlap(b0) & (pos > i)
            suppv[pl.ds(b0, L)] = jnp.maximum(
                suppv[pl.ds(b0, L)], ns0.astype(jnp.float32))

            @plsc.parallel_loop(k0 + 1, nch, unroll=2)
            def _rest(k):
                b = k * L
                ns = overlap(b)
                suppv[pl.ds(b, L)] = jnp.maximum(
                    suppv[pl.ds(b, L)], ns.astype(jnp.float32))

        return 0

    jax.lax.fori_loop(0, n, outer, 0)

    def kval_body(k, _):
        r = k // 8
        c = (k % 8) * L
        kval2[r, pl.ds(c, L)] = 1.0 - suppv[pl.ds(k * L, L)]
        return 0

    jax.lax.fori_loop(0, nch, kval_body, 0)

    def scatter_body(j, _):
        pltpu.sync_copy(kval2.at[j], out_ref.at[lidx2.at[j]])
        return 0

    jax.lax.fori_loop(0, NROWS, scatter_body, 0)


def _mask_kernel(b_ref, ssel_ref, keep_ref, bo_ref, so_ref):
    k = keep_ref[...]
    bo_ref[...] = b_ref[...] * k
    so_ref[...] = ssel_ref[...] * k


@jax.jit
def kernel(boxes, scores):
    grid = N // BLK
    cls_c, ssel_c, key_c = pl.pallas_call(
        _cls_kernel,
        grid=(grid,),
        in_specs=[pl.BlockSpec((BLK, NUM_CLASSES), lambda i: (i, 0))],
        out_specs=[
            pl.BlockSpec((BLK, 1), lambda i: (i, 0)),
            pl.BlockSpec((BLK, 1), lambda i: (i, 0)),
            pl.BlockSpec((BLK, 1), lambda i: (i, 0)),
        ],
        out_shape=[
            jax.ShapeDtypeStruct((N, 1), jnp.int32),
            jax.ShapeDtypeStruct((N, 1), jnp.float32),
            jax.ShapeDtypeStruct((N, 1), jnp.float32),
        ],
    )(scores)
    cls_inds = cls_c[:, 0]

    order = jnp.argsort(key_c[:, 0]).astype(jnp.int32)
    order_p = jnp.concatenate([order, jnp.zeros((NP - N,), jnp.int32)])
    counts = jnp.sum(
        (cls_inds[None, :] == jnp.arange(NUM_CLASSES, dtype=jnp.int32)[:, None])
        .astype(jnp.int32), axis=1)
    starts = jnp.concatenate(
        [jnp.zeros((1,), jnp.int32), jnp.cumsum(counts)[:-1].astype(jnp.int32)])
    starts_p = jnp.concatenate(
        [starts, jnp.full((128 - NUM_CLASSES,), N, jnp.int32)])
    counts_p = jnp.concatenate(
        [counts, jnp.zeros((128 - NUM_CLASSES,), jnp.int32)])

    mesh = plsc.VectorSubcoreMesh(core_axis_name="c", subcore_axis_name="s")
    keep_p = pl.kernel(
        _sc_nms,
        out_type=jax.ShapeDtypeStruct((NP,), jnp.float32),
        mesh=mesh,
        compiler_params=pltpu.CompilerParams(needs_layout_passes=False),
        scratch_types=[
            pltpu.VMEM((4 * N,), jnp.float32),
            pltpu.VMEM((NP,), jnp.int32),
            pltpu.VMEM((128,), jnp.int32),
            pltpu.VMEM((128,), jnp.int32),
            pltpu.VMEM((NP,), jnp.float32),
            pltpu.VMEM((NP,), jnp.float32),
            pltpu.VMEM((NP,), jnp.float32),
            pltpu.VMEM((NP,), jnp.float32),
            pltpu.VMEM((NP,), jnp.float32),
            pltpu.VMEM((NP,), jnp.float32),
            pltpu.VMEM((NROWS, 128), jnp.int32),
            pltpu.VMEM((NROWS, 128), jnp.float32),
        ],
    )(boxes.reshape(-1), order_p, starts_p, counts_p)

    keep_col = keep_p[:N].reshape(N, 1)

    bo, so = pl.pallas_call(
        _mask_kernel,
        grid=(grid,),
        in_specs=[
            pl.BlockSpec((BLK, 4), lambda i: (i, 0)),
            pl.BlockSpec((BLK, 1), lambda i: (i, 0)),
            pl.BlockSpec((BLK, 1), lambda i: (i, 0)),
        ],
        out_specs=[
            pl.BlockSpec((BLK, 4), lambda i: (i, 0)),
            pl.BlockSpec((BLK, 1), lambda i: (i, 0)),
        ],
        out_shape=[
            jax.ShapeDtypeStruct((N, 4), jnp.float32),
            jax.ShapeDtypeStruct((N, 1), jnp.float32),
        ],
    )(boxes, ssel_c, keep_col)

    return bo, so[:, 0], cls_inds


# speedup vs baseline: 259.7141x; 259.7141x over previous
import functools

import jax
import jax.numpy as jnp
from jax.experimental import pallas as pl
from jax.experimental.pallas import tpu as pltpu
from jax.experimental.pallas import tpu_sc as plsc

N = 5000
NUM_CLASSES = 20
NP = 5120
L = 16
NTILES = 32
NCHUNKS = NP // L
NROWS = 40
THRESH = 0.5
BLK = 1000


def _cls_kernel(s_ref, cls_ref, ssel_ref, key_ref):
    s = s_ref[...]
    m = jnp.max(s, axis=1, keepdims=True)
    lane = jax.lax.broadcasted_iota(jnp.int32, s.shape, 1)
    idx = jnp.min(jnp.where(s == m, lane, NUM_CLASSES), axis=1, keepdims=True)
    cls_ref[...] = idx
    ssel_ref[...] = m
    key_ref[...] = idx.astype(jnp.float32) * 2.0 - m


def _sc_nms(bh, ordh, sth, cnth, out_ref,
            bfv, ordv, stv, cntv,
            lx1, ly1, lx2, ly2, lar, suppv, lidx, keeprow):
    wid = jax.lax.axis_index("s") * 2 + jax.lax.axis_index("c")
    iota = jax.lax.iota(jnp.int32, L)

    def zero_body(k, _):
        keeprow[pl.ds(k * L, L)] = jnp.zeros((L,), jnp.float32)
        return 0

    jax.lax.fori_loop(0, NCHUNKS, zero_body, 0)

    pltpu.sync_copy(bh, bfv)
    pltpu.sync_copy(ordh, ordv)
    pltpu.sync_copy(sth, stv)
    pltpu.sync_copy(cnth, cntv)

    def sload(ref, i):
        v = plsc.load_gather(ref, [jnp.full((L,), i, jnp.int32)])
        return v[0]

    start = sload(stv, wid)
    n = sload(cntv, wid)
    nch = (n + L - 1) // L

    def gather_body(k, _):
        p16 = jnp.full((L,), start + k * L, jnp.int32) + iota
        idx16 = plsc.load_gather(ordv, [p16])
        lidx[pl.ds(k * L, L)] = idx16
        i4 = idx16 * 4
        a = plsc.load_gather(bfv, [i4])
        b = plsc.load_gather(bfv, [i4 + 1])
        cc = plsc.load_gather(bfv, [i4 + 2])
        d = plsc.load_gather(bfv, [i4 + 3])
        lx1[pl.ds(k * L, L)] = a
        ly1[pl.ds(k * L, L)] = b
        lx2[pl.ds(k * L, L)] = cc
        ly2[pl.ds(k * L, L)] = d
        lar[pl.ds(k * L, L)] = (cc - a) * (d - b)
        suppv[pl.ds(k * L, L)] = jnp.zeros((L,), jnp.float32)
        return 0

    jax.lax.fori_loop(0, nch, gather_body, 0)

    def outer(i, _):
        ii = jnp.full((L,), i, jnp.int32)
        si = plsc.load_gather(suppv, [ii])
        act = si[0] == 0.0

        @pl.when(act)
        def _sweep():
            x1i = plsc.load_gather(lx1, [ii])
            y1i = plsc.load_gather(ly1, [ii])
            x2i = plsc.load_gather(lx2, [ii])
            y2i = plsc.load_gather(ly2, [ii])
            ai = plsc.load_gather(lar, [ii])

            def overlap(b0):
                xx1 = jnp.maximum(lx1[pl.ds(b0, L)], x1i)
                yy1 = jnp.maximum(ly1[pl.ds(b0, L)], y1i)
                xx2 = jnp.minimum(lx2[pl.ds(b0, L)], x2i)
                yy2 = jnp.minimum(ly2[pl.ds(b0, L)], y2i)
                w = jnp.maximum(1e-10, xx2 - xx1)
                h = jnp.maximum(1e-10, yy2 - yy1)
                inter = w * h
                denom = (ai + lar[pl.ds(b0, L)]) - inter
                return (inter > THRESH * denom) & (denom >= 0.0)

            k0 = i // L
            b0 = k0 * L
            pos = jnp.full((L,), b0, jnp.int32) + iota
            ns0 = overlap(b0) & (pos > i)
            suppv[pl.ds(b0, L)] = jnp.maximum(
                suppv[pl.ds(b0, L)], ns0.astype(jnp.float32))

            @plsc.parallel_loop(k0 + 1, nch, unroll=2)
            def _rest(k):
                b = k * L
                ns = overlap(b)
                suppv[pl.ds(b, L)] = jnp.maximum(
                    suppv[pl.ds(b, L)], ns.astype(jnp.float32))

        return 0

    jax.lax.fori_loop(0, n, outer, 0)

    def scatter_body(k, _):
        idx16 = lidx[pl.ds(k * L, L)]
        sp = suppv[pl.ds(k * L, L)]
        pos = jnp.full((L,), k * L, jnp.int32) + iota
        m = pos < n
        plsc.store_scatter(keeprow, [idx16], 1.0 - sp, mask=m)
        return 0

    jax.lax.fori_loop(0, nch, scatter_body, 0)

    pltpu.sync_copy(keeprow, out_ref.at[wid])


def _mask_kernel(b_ref, ssel_ref, keep_ref, bo_ref, so_ref):
    k = keep_ref[...]
    bo_ref[...] = b_ref[...] * k
    so_ref[...] = ssel_ref[...] * k


@jax.jit
def kernel(boxes, scores):
    grid = N // BLK
    cls_c, ssel_c, key_c = pl.pallas_call(
        _cls_kernel,
        grid=(grid,),
        in_specs=[pl.BlockSpec((BLK, NUM_CLASSES), lambda i: (i, 0))],
        out_specs=[
            pl.BlockSpec((BLK, 1), lambda i: (i, 0)),
            pl.BlockSpec((BLK, 1), lambda i: (i, 0)),
            pl.BlockSpec((BLK, 1), lambda i: (i, 0)),
        ],
        out_shape=[
            jax.ShapeDtypeStruct((N, 1), jnp.int32),
            jax.ShapeDtypeStruct((N, 1), jnp.float32),
            jax.ShapeDtypeStruct((N, 1), jnp.float32),
        ],
    )(scores)
    cls_inds = cls_c[:, 0]

    order = jnp.argsort(key_c[:, 0]).astype(jnp.int32)
    order_p = jnp.concatenate([order, jnp.zeros((NP - N,), jnp.int32)])
    counts = jnp.sum(
        (cls_inds[None, :] == jnp.arange(NUM_CLASSES, dtype=jnp.int32)[:, None])
        .astype(jnp.int32), axis=1)
    starts = jnp.concatenate(
        [jnp.zeros((1,), jnp.int32), jnp.cumsum(counts)[:-1].astype(jnp.int32)])
    starts_p = jnp.concatenate(
        [starts, jnp.full((128 - NUM_CLASSES,), N, jnp.int32)])
    counts_p = jnp.concatenate(
        [counts, jnp.zeros((128 - NUM_CLASSES,), jnp.int32)])

    mesh = plsc.VectorSubcoreMesh(core_axis_name="c", subcore_axis_name="s")
    keep_rows = pl.kernel(
        _sc_nms,
        out_type=jax.ShapeDtypeStruct((NTILES, NP), jnp.float32),
        mesh=mesh,
        compiler_params=pltpu.CompilerParams(needs_layout_passes=False),
        scratch_types=[
            pltpu.VMEM((4 * N,), jnp.float32),
            pltpu.VMEM((NP,), jnp.int32),
            pltpu.VMEM((128,), jnp.int32),
            pltpu.VMEM((128,), jnp.int32),
            pltpu.VMEM((NP,), jnp.float32),
            pltpu.VMEM((NP,), jnp.float32),
            pltpu.VMEM((NP,), jnp.float32),
            pltpu.VMEM((NP,), jnp.float32),
            pltpu.VMEM((NP,), jnp.float32),
            pltpu.VMEM((NP,), jnp.float32),
            pltpu.VMEM((NP,), jnp.int32),
            pltpu.VMEM((NP,), jnp.float32),
        ],
    )(boxes.reshape(-1), order_p, starts_p, counts_p)

    keep_col = jnp.max(keep_rows, axis=0)[:N].reshape(N, 1)

    bo, so = pl.pallas_call(
        _mask_kernel,
        grid=(grid,),
        in_specs=[
            pl.BlockSpec((BLK, 4), lambda i: (i, 0)),
            pl.BlockSpec((BLK, 1), lambda i: (i, 0)),
            pl.BlockSpec((BLK, 1), lambda i: (i, 0)),
        ],
        out_specs=[
            pl.BlockSpec((BLK, 4), lambda i: (i, 0)),
            pl.BlockSpec((BLK, 1), lambda i: (i, 0)),
        ],
        out_shape=[
            jax.ShapeDtypeStruct((N, 4), jnp.float32),
            jax.ShapeDtypeStruct((N, 1), jnp.float32),
        ],
    )(boxes, ssel_c, keep_col)

    return bo, so[:, 0], cls_inds


# speedup vs baseline: 322.0578x; 1.2400x over previous
import functools

import jax
import jax.numpy as jnp
from jax.experimental import pallas as pl
from jax.experimental.pallas import tpu as pltpu
from jax.experimental.pallas import tpu_sc as plsc

N = 5000
NUM_CLASSES = 20
NP = 5120
L = 16
NTILES = 32
NCHUNKS = NP // L
NROWS = 40
THRESH = 0.5
BLK = 1000


def _cls_kernel(st_ref, cls_ref, ssel_ref, key_ref, cnt_ref):
    s = st_ref[...]
    m = jnp.max(s, axis=0, keepdims=True)
    row = jax.lax.broadcasted_iota(jnp.int32, s.shape, 0)
    idx = jnp.min(jnp.where(s == m, row, NUM_CLASSES), axis=0, keepdims=True)
    cls_ref[...] = idx
    ssel_ref[...] = m
    key_ref[...] = idx.astype(jnp.float32) * 2.0 - m
    valid = jax.lax.broadcasted_iota(jnp.int32, s.shape, 1) < N
    onehot = (row == idx) & valid
    cnt_ref[...] = jnp.sum(onehot.astype(jnp.int32), axis=1, keepdims=True)


def _sc_nms(bh, ordh, sth, cnth, out_ref,
            bfv, ordv, stv, cntv,
            lx1, ly1, lx2, ly2, lar, suppv, lidx, keeprow):
    wid = jax.lax.axis_index("s") * 2 + jax.lax.axis_index("c")
    iota = jax.lax.iota(jnp.int32, L)

    def zero_body(k, _):
        keeprow[pl.ds(k * L, L)] = jnp.zeros((L,), jnp.float32)
        return 0

    jax.lax.fori_loop(0, NCHUNKS, zero_body, 0)

    pltpu.sync_copy(bh, bfv)
    pltpu.sync_copy(ordh, ordv)
    pltpu.sync_copy(sth, stv)
    pltpu.sync_copy(cnth, cntv)

    def sload(ref, i):
        v = plsc.load_gather(ref, [jnp.full((L,), i, jnp.int32)])
        return v[0]

    start = sload(stv, wid)
    n = sload(cntv, wid)
    nch = (n + L - 1) // L

    def gather_body(k, _):
        p16 = jnp.full((L,), start + k * L, jnp.int32) + iota
        idx16 = plsc.load_gather(ordv, [p16])
        lidx[pl.ds(k * L, L)] = idx16
        i4 = idx16 * 4
        a = plsc.load_gather(bfv, [i4])
        b = plsc.load_gather(bfv, [i4 + 1])
        cc = plsc.load_gather(bfv, [i4 + 2])
        d = plsc.load_gather(bfv, [i4 + 3])
        lx1[pl.ds(k * L, L)] = a
        ly1[pl.ds(k * L, L)] = b
        lx2[pl.ds(k * L, L)] = cc
        ly2[pl.ds(k * L, L)] = d
        lar[pl.ds(k * L, L)] = (cc - a) * (d - b)
        suppv[pl.ds(k * L, L)] = jnp.zeros((L,), jnp.float32)
        return 0

    jax.lax.fori_loop(0, nch, gather_body, 0)

    def outer(i, _):
        ii = jnp.full((L,), i, jnp.int32)
        si = plsc.load_gather(suppv, [ii])
        act = si[0] == 0.0

        @pl.when(act)
        def _sweep():
            x1i = plsc.load_gather(lx1, [ii])
            y1i = plsc.load_gather(ly1, [ii])
            x2i = plsc.load_gather(lx2, [ii])
            y2i = plsc.load_gather(ly2, [ii])
            ai = plsc.load_gather(lar, [ii])

            def overlap(b0):
                xx1 = jnp.maximum(lx1[pl.ds(b0, L)], x1i)
                yy1 = jnp.maximum(ly1[pl.ds(b0, L)], y1i)
                xx2 = jnp.minimum(lx2[pl.ds(b0, L)], x2i)
                yy2 = jnp.minimum(ly2[pl.ds(b0, L)], y2i)
                w = jnp.maximum(1e-10, xx2 - xx1)
                h = jnp.maximum(1e-10, yy2 - yy1)
                inter = w * h
                denom = (ai + lar[pl.ds(b0, L)]) - inter
                return (inter > THRESH * denom) & (denom >= 0.0)

            k0 = i // L
            b0 = k0 * L
            pos = jnp.full((L,), b0, jnp.int32) + iota
            ns0 = overlap(b0) & (pos > i)
            suppv[pl.ds(b0, L)] = jnp.maximum(
                suppv[pl.ds(b0, L)], ns0.astype(jnp.float32))

            @plsc.parallel_loop(k0 + 1, nch, unroll=2)
            def _rest(k):
                b = k * L
                ns = overlap(b)
                suppv[pl.ds(b, L)] = jnp.maximum(
                    suppv[pl.ds(b, L)], ns.astype(jnp.float32))

        return 0

    jax.lax.fori_loop(0, n, outer, 0)

    def scatter_body(k, _):
        idx16 = lidx[pl.ds(k * L, L)]
        sp = suppv[pl.ds(k * L, L)]
        pos = jnp.full((L,), k * L, jnp.int32) + iota
        m = pos < n
        plsc.store_scatter(keeprow, [idx16], 1.0 - sp, mask=m)
        return 0

    jax.lax.fori_loop(0, nch, scatter_body, 0)

    pltpu.sync_copy(keeprow, out_ref.at[wid])


def _combine_kernel(rows_ref, bt_ref, ssel_ref, bo_ref, so_ref):
    keep = jnp.max(rows_ref[...], axis=0, keepdims=True)
    bo_ref[...] = bt_ref[...] * keep
    so_ref[...] = ssel_ref[...] * keep


@jax.jit
def kernel(boxes, scores):
    st = jnp.zeros((NUM_CLASSES, NP), jnp.float32)
    st = st.at[:, :N].set(scores.T)
    cls_p, ssel_p, key_p, cnt_p = pl.pallas_call(
        _cls_kernel,
        out_shape=[
            jax.ShapeDtypeStruct((1, NP), jnp.int32),
            jax.ShapeDtypeStruct((1, NP), jnp.float32),
            jax.ShapeDtypeStruct((1, NP), jnp.float32),
            jax.ShapeDtypeStruct((NUM_CLASSES, 1), jnp.int32),
        ],
    )(st)
    cls_inds = cls_p[0, :N]

    order = jnp.argsort(key_p[0, :N]).astype(jnp.int32)
    order_p = jnp.concatenate([order, jnp.zeros((NP - N,), jnp.int32)])
    counts = cnt_p[:, 0]
    starts = jnp.concatenate(
        [jnp.zeros((1,), jnp.int32), jnp.cumsum(counts)[:-1].astype(jnp.int32)])
    starts_p = jnp.concatenate(
        [starts, jnp.full((128 - NUM_CLASSES,), N, jnp.int32)])
    counts_p = jnp.concatenate(
        [counts, jnp.zeros((128 - NUM_CLASSES,), jnp.int32)])

    mesh = plsc.VectorSubcoreMesh(core_axis_name="c", subcore_axis_name="s")
    keep_rows = pl.kernel(
        _sc_nms,
        out_type=jax.ShapeDtypeStruct((NTILES, NP), jnp.float32),
        mesh=mesh,
        compiler_params=pltpu.CompilerParams(needs_layout_passes=False),
        scratch_types=[
            pltpu.VMEM((4 * N,), jnp.float32),
            pltpu.VMEM((NP,), jnp.int32),
            pltpu.VMEM((128,), jnp.int32),
            pltpu.VMEM((128,), jnp.int32),
            pltpu.VMEM((NP,), jnp.float32),
            pltpu.VMEM((NP,), jnp.float32),
            pltpu.VMEM((NP,), jnp.float32),
            pltpu.VMEM((NP,), jnp.float32),
            pltpu.VMEM((NP,), jnp.float32),
            pltpu.VMEM((NP,), jnp.float32),
            pltpu.VMEM((NP,), jnp.int32),
            pltpu.VMEM((NP,), jnp.float32),
        ],
    )(boxes.reshape(-1), order_p, starts_p, counts_p)

    bt = jnp.zeros((4, NP), jnp.float32)
    bt = bt.at[:, :N].set(boxes.T)
    bo, so = pl.pallas_call(
        _combine_kernel,
        out_shape=[
            jax.ShapeDtypeStruct((4, NP), jnp.float32),
            jax.ShapeDtypeStruct((1, NP), jnp.float32),
        ],
    )(keep_rows, bt, ssel_p)

    boxes_out = bo[:, :N].T
    scores_out = so[0, :N]
    return boxes_out, scores_out, cls_inds
